# baseline (speedup 1.0000x reference)
import functools

import jax
import jax.numpy as jnp
from jax import lax
from jax.experimental import pallas as pl
from jax.experimental.pallas import tpu as pltpu
from jax.experimental.pallas import tpu_sc as plsc

N = 10000
NP_ = 10240
E = 320000
NC = 2
NS = 16
K = 125
CH = E // (NC * NS * K)
NPT = NP_ // NS
HL = 16
BLK = 1000
GB = N // BLK

_SC_MESH = dict(core_axis_name="c", subcore_axis_name="s")
_SC_PARAMS = pltpu.CompilerParams(use_tc_tiling_on_sc=False)


@functools.partial(
    pl.kernel,
    mesh=plsc.VectorSubcoreMesh(**_SC_MESH),
    compiler_params=_SC_PARAMS,
    out_type=jax.ShapeDtypeStruct((NC, NP_, HL), jnp.float32),
    scratch_types=[
        pltpu.VMEM((CH, K), jnp.int32),
        pltpu.VMEM((128, HL), jnp.float32),
        pltpu.VMEM_SHARED((NP_, HL), jnp.float32),
    ],
)
def _hist(ei_hbm, out_hbm, idx_v, ones_v, acc):
    c = lax.axis_index("c")
    s = lax.axis_index("s")
    pltpu.sync_copy(ei_hbm.at[1, c, s], idx_v)

    def fill(i, carry):
        ones_v[i, :] = jnp.full((16,), 1.0, jnp.float32)
        return carry

    lax.fori_loop(0, 128, fill, 0)
    for b in range(NPT // 128):
        pltpu.sync_copy(ones_v, acc.at[pl.ds(s * NPT + b * 128, 128)])
    plsc.subcore_barrier()

    def body(j, carry):
        pltpu.sync_copy(ones_v.at[pl.ds(0, K)], acc.at[idx_v.at[j]], add=True)
        return carry

    lax.fori_loop(0, CH, body, 0)
    plsc.subcore_barrier()
    pltpu.sync_copy(acc.at[pl.ds(s * NPT, NPT)], out_hbm.at[c, pl.ds(s * NPT, NPT)])


def _make_prop(D):
    def prop_kernel(ei_hbm, g_hbm, out_hbm, srcv, dstv, rows_a, rows_b,
                    acc, gsem_a, gsem_b, ssem_a, ssem_b):
        c = lax.axis_index("c")
        s = lax.axis_index("s")
        pltpu.sync_copy(ei_hbm.at[0, c, s], srcv)
        pltpu.sync_copy(ei_hbm.at[1, c, s], dstv)
        pltpu.sync_copy(g_hbm.at[pl.ds(s * NPT, NPT)], acc.at[pl.ds(s * NPT, NPT)])
        plsc.subcore_barrier()

        pltpu.async_copy(g_hbm.at[srcv.at[0]], rows_a, gsem_a)
        pltpu.async_copy(g_hbm.at[srcv.at[1]], rows_b, gsem_b)

        def body(i, carry):
            ja = 2 * i
            jb = 2 * i + 1
            pltpu.make_async_copy(g_hbm.at[srcv.at[ja]], rows_a, gsem_a).wait()
            pltpu.async_copy(rows_a, acc.at[dstv.at[ja]], ssem_a, add=True)
            pltpu.make_async_copy(g_hbm.at[srcv.at[jb]], rows_b, gsem_b).wait()
            pltpu.async_copy(rows_b, acc.at[dstv.at[jb]], ssem_b, add=True)

            @pl.when(ja + 2 < CH)
            def _():
                pltpu.make_async_copy(rows_a, acc.at[dstv.at[ja]], ssem_a).wait()
                pltpu.async_copy(g_hbm.at[srcv.at[ja + 2]], rows_a, gsem_a)
                pltpu.make_async_copy(rows_b, acc.at[dstv.at[jb]], ssem_b).wait()
                pltpu.async_copy(g_hbm.at[srcv.at[jb + 2]], rows_b, gsem_b)

            return carry

        lax.fori_loop(0, CH // 2, body, 0)
        pltpu.make_async_copy(rows_a, acc.at[dstv.at[0]], ssem_a).wait()
        pltpu.make_async_copy(rows_b, acc.at[dstv.at[0]], ssem_b).wait()
        plsc.subcore_barrier()
        pltpu.sync_copy(acc.at[pl.ds(s * NPT, NPT)],
                        out_hbm.at[c, pl.ds(s * NPT, NPT)])

    return pl.kernel(
        prop_kernel,
        mesh=plsc.VectorSubcoreMesh(**_SC_MESH),
        compiler_params=_SC_PARAMS,
        out_type=jax.ShapeDtypeStruct((NC, NP_, D), jnp.float32),
        scratch_types=[
            pltpu.VMEM((CH, K), jnp.int32),
            pltpu.VMEM((CH, K), jnp.int32),
            pltpu.VMEM((K, D), jnp.float32),
            pltpu.VMEM((K, D), jnp.float32),
            pltpu.VMEM_SHARED((NP_, D), jnp.float32),
            pltpu.SemaphoreType.DMA,
            pltpu.SemaphoreType.DMA,
            pltpu.SemaphoreType.DMA,
            pltpu.SemaphoreType.DMA,
        ],
    )


_prop64 = _make_prop(64)
_prop48 = _make_prop(48)


_DN = (((1,), (1,)), ((), ()))


def _dot(a, b):
    return lax.dot_general(a, b, _DN, preferred_element_type=jnp.float32)


def _kan_lhs(xb):
    u = 2.0 * xb + 5.0
    t = jnp.floor(u)
    f = u - t
    f2 = f * f
    f3 = f2 * f
    sixth = jnp.float32(1.0 / 6.0)
    v0 = f3 * sixth
    v1 = (-3.0 * f3 + 3.0 * f2 + 3.0 * f + 1.0) * sixth
    v2 = (3.0 * f3 - 6.0 * f2 + 4.0) * sixth
    omf = 1.0 - f
    v3 = omf * omf * omf * sixth
    cols = [jax.nn.silu(xb)]
    for c in range(7):
        fc = jnp.float32(c)
        cols.append(jnp.where(t == fc, v0, 0.0)
                    + jnp.where(t == fc + 1.0, v1, 0.0)
                    + jnp.where(t == fc + 2.0, v2, 0.0)
                    + jnp.where(t == fc + 3.0, v3, 0.0))
    return jnp.concatenate(cols, axis=1)


def _kan_rhs(wb, ws_t, scaler):
    return jnp.concatenate([wb] + [ws_t[c] * scaler for c in range(7)], axis=1)


def _dinv_of(cnt):
    return lax.rsqrt(cnt[0, :, 0:1] + cnt[1, :, 0:1] - 1.0)


def _kan1_body(x_ref, w1b_ref, w1s_ref, w1c_ref, w2b_ref, w2s_ref, w2c_ref,
               h_ref, hx_ref):
    lhs = _kan_lhs(x_ref[...])
    rhs = jnp.concatenate(
        [_kan_rhs(w1b_ref[...], w1s_ref, w1c_ref[...]),
         _kan_rhs(w2b_ref[...], w2s_ref, w2c_ref[...])], axis=0)
    out = _dot(lhs, rhs)
    h_ref[...] = out[:, :64]
    hx_ref[...] = out[:, 64:104]


def _scale_body(h_ref, cnt_ref, g_ref):
    g_ref[...] = h_ref[...] * _dinv_of(cnt_ref[...])


def _c1_body(s1_ref, g1_ref, cnt_ref, b1_ref, st_ref):
    i = pl.program_id(0)
    s1 = s1_ref[...]
    dinv = _dinv_of(cnt_ref[...])
    z = dinv * (s1[0] + s1[1] - g1_ref[...]) + b1_ref[...]

    @pl.when(i == 0)
    def _():
        st_ref[...] = jnp.zeros_like(st_ref)

    st_ref[...] = st_ref[...] + jnp.concatenate(
        [z.sum(0)[None, :], (z * z).sum(0)[None, :]], axis=0)


def _c2_body(s1_ref, g1_ref, cnt_ref, b1_ref, st_ref, gam_ref, bet_ref,
             hx_ref, w2b_ref, w2s_ref, w2c_ref, h_ref, g_ref):
    s1 = s1_ref[...]
    dinv = _dinv_of(cnt_ref[...])
    a1 = dinv * (s1[0] + s1[1] - g1_ref[...]) + b1_ref[...]
    st = st_ref[...]
    mean = st[0:1] * jnp.float32(1.0 / N)
    var = st[1:2] * jnp.float32(1.0 / N) - mean * mean
    hb = (a1 - mean) * lax.rsqrt(var + 1e-5) * gam_ref[...] + bet_ref[...]
    h = hx_ref[...] + _dot(_kan_lhs(hb),
                           _kan_rhs(w2b_ref[...], w2s_ref, w2c_ref[...]))
    h_ref[...] = h
    g_ref[...] = jnp.concatenate(
        [h * dinv, jnp.zeros((h.shape[0], 8), jnp.float32)], axis=1)


def _c3_body(s2_ref, g2_ref, cnt_ref, bo_ref, out_ref):
    s2 = s2_ref[...]
    dinv = _dinv_of(cnt_ref[...])
    out_ref[...] = (dinv * (s2[0] + s2[1] - g2_ref[...]))[:, :40] + bo_ref[...]


def _full(shape):
    return pl.BlockSpec(shape, lambda i: tuple(0 for _ in shape))


def _rows(d):
    return pl.BlockSpec((BLK, d), lambda i: (i, 0))


def _rows3(d):
    return pl.BlockSpec((NC, BLK, d), lambda i: (0, i, 0))


_CNT_SPEC = _rows3(HL)


def _padded(d):
    return jax.ShapeDtypeStruct((NP_, d), jnp.float32)


def _kan1_call(x, w1b, w1s_t, w1c, w2b, w2s_t, w2c):
    return pl.pallas_call(
        _kan1_body,
        grid=(GB,),
        in_specs=[_rows(128), _full((64, 128)), _full((7, 64, 128)),
                  _full((64, 128)), _full((40, 128)), _full((7, 40, 128)),
                  _full((40, 128))],
        out_specs=[_rows(64), _rows(40)],
        out_shape=[jax.ShapeDtypeStruct((N, 64), jnp.float32),
                   jax.ShapeDtypeStruct((N, 40), jnp.float32)],
    )(x, w1b, w1s_t, w1c, w2b, w2s_t, w2c)


def _scale_call(h1, hist):
    return pl.pallas_call(
        _scale_body,
        grid=(GB,),
        in_specs=[_rows(64), _CNT_SPEC],
        out_specs=_rows(64),
        out_shape=_padded(64),
    )(h1, hist)


def _c1_call(s1, g1, hist, b1):
    return pl.pallas_call(
        _c1_body,
        grid=(GB,),
        in_specs=[_rows3(64), _rows(64), _CNT_SPEC, _full((1, 64))],
        out_specs=_full((2, 64)),
        out_shape=jax.ShapeDtypeStruct((2, 64), jnp.float32),
    )(s1, g1, hist, b1)


def _c2_call(s1, g1, hist, b1, st, gam, bet, h2x, w2b, w2s_t, w2c):
    return pl.pallas_call(
        _c2_body,
        grid=(GB,),
        in_specs=[_rows3(64), _rows(64), _CNT_SPEC, _full((1, 64)),
                  _full((2, 64)), _full((1, 64)), _full((1, 64)), _rows(40),
                  _full((40, 64)), _full((7, 40, 64)), _full((40, 64))],
        out_specs=[_rows(40), _rows(48)],
        out_shape=[jax.ShapeDtypeStruct((N, 40), jnp.float32), _padded(48)],
    )(s1, g1, hist, b1, st, gam, bet, h2x, w2b, w2s_t, w2c)


def _c3_call(s2, g2, hist, bo):
    return pl.pallas_call(
        _c3_body,
        grid=(GB,),
        in_specs=[_rows3(48), _rows(48), _CNT_SPEC, _full((1, 40))],
        out_specs=_rows(40),
        out_shape=jax.ShapeDtypeStruct((N, 40), jnp.float32),
    )(s2, g2, hist, bo)


def kernel(x, edge_index, w1_base, w1_spline, w1_scaler, grid1, b1,
           bn_gamma, bn_beta, wo_base, wo_spline, wo_scaler, grido, bo):
    ei = edge_index.astype(jnp.int32).reshape(2, NC, NS, CH, K)
    w1_sp_t = jnp.transpose(w1_spline, (2, 0, 1))
    wo_sp_t = jnp.transpose(wo_spline, (2, 0, 1))

    hist = _hist(ei)
    h1, h2x = _kan1_call(x, w1_base, w1_sp_t, w1_scaler,
                         wo_base[:, :128], wo_sp_t[:, :, :128],
                         wo_scaler[:, :128])
    g1 = _scale_call(h1, hist)
    s1 = _prop64(ei, g1)
    st = _c1_call(s1, g1, hist, b1.reshape(1, 64))
    h2, g2 = _c2_call(s1, g1, hist, b1.reshape(1, 64), st,
                      bn_gamma.reshape(1, 64), bn_beta.reshape(1, 64), h2x,
                      wo_base[:, 128:], wo_sp_t[:, :, 128:], wo_scaler[:, 128:])
    s2 = _prop48(ei, g2)
    return _c3_call(s2, g2, hist, bo.reshape(1, 40))


# speedup vs baseline: 1.0951x; 1.0951x over previous
import functools

import jax
import jax.numpy as jnp
from jax import lax
from jax.experimental import pallas as pl
from jax.experimental.pallas import tpu as pltpu
from jax.experimental.pallas import tpu_sc as plsc

N = 10000
NP_ = 10240
E = 320000
NC = 2
NS = 16
K = 125
CH = E // (NC * NS * K)
NPT = NP_ // NS
HL = 16
BLK = 1000
GB = N // BLK

_SC_MESH = dict(core_axis_name="c", subcore_axis_name="s")
_SC_PARAMS = pltpu.CompilerParams(use_tc_tiling_on_sc=False)


@functools.partial(
    pl.kernel,
    mesh=plsc.VectorSubcoreMesh(**_SC_MESH),
    compiler_params=_SC_PARAMS,
    out_type=jax.ShapeDtypeStruct((NC, NP_, HL), jnp.float32),
    scratch_types=[
        pltpu.VMEM((CH, K), jnp.int32),
        pltpu.VMEM((128, HL), jnp.float32),
        pltpu.VMEM_SHARED((NP_, HL), jnp.float32),
    ],
)
def _hist(ei_hbm, out_hbm, idx_v, ones_v, acc):
    c = lax.axis_index("c")
    s = lax.axis_index("s")
    pltpu.sync_copy(ei_hbm.at[1, c, s], idx_v)

    def fill(i, carry):
        ones_v[i, :] = jnp.full((16,), 1.0, jnp.float32)
        return carry

    lax.fori_loop(0, 128, fill, 0)
    for b in range(NPT // 128):
        pltpu.sync_copy(ones_v, acc.at[pl.ds(s * NPT + b * 128, 128)])
    plsc.subcore_barrier()

    def body(j, carry):
        pltpu.sync_copy(ones_v.at[pl.ds(0, K)], acc.at[idx_v.at[j]], add=True)
        return carry

    lax.fori_loop(0, CH, body, 0)
    plsc.subcore_barrier()
    pltpu.sync_copy(acc.at[pl.ds(s * NPT, NPT)], out_hbm.at[c, pl.ds(s * NPT, NPT)])


def _make_prop(D):
    def prop_kernel(ei_hbm, g_hbm, out_hbm, srcv, dstv, rows_a, rows_b,
                    acc, gsem_a, gsem_b):
        c = lax.axis_index("c")
        s = lax.axis_index("s")
        pltpu.sync_copy(ei_hbm.at[0, c, s], srcv)
        pltpu.sync_copy(ei_hbm.at[1, c, s], dstv)
        pltpu.sync_copy(g_hbm.at[pl.ds(s * NPT, NPT)], acc.at[pl.ds(s * NPT, NPT)])
        plsc.subcore_barrier()

        pltpu.async_copy(g_hbm.at[srcv.at[0]], rows_a, gsem_a)

        def body(i, carry):
            ja = 2 * i
            jb = 2 * i + 1
            cp_b = pltpu.async_copy(g_hbm.at[srcv.at[jb]], rows_b, gsem_b)
            pltpu.make_async_copy(g_hbm.at[srcv.at[ja]], rows_a, gsem_a).wait()
            pltpu.sync_copy(rows_a, acc.at[dstv.at[ja]], add=True)

            @pl.when(jb + 1 < CH)
            def _():
                pltpu.async_copy(g_hbm.at[srcv.at[jb + 1]], rows_a, gsem_a)

            cp_b.wait()
            pltpu.sync_copy(rows_b, acc.at[dstv.at[jb]], add=True)
            return carry

        lax.fori_loop(0, CH // 2, body, 0)
        plsc.subcore_barrier()
        pltpu.sync_copy(acc.at[pl.ds(s * NPT, NPT)],
                        out_hbm.at[c, pl.ds(s * NPT, NPT)])

    return pl.kernel(
        prop_kernel,
        mesh=plsc.VectorSubcoreMesh(**_SC_MESH),
        compiler_params=_SC_PARAMS,
        out_type=jax.ShapeDtypeStruct((NC, NP_, D), jnp.float32),
        scratch_types=[
            pltpu.VMEM((CH, K), jnp.int32),
            pltpu.VMEM((CH, K), jnp.int32),
            pltpu.VMEM((K, D), jnp.float32),
            pltpu.VMEM((K, D), jnp.float32),
            pltpu.VMEM_SHARED((NP_, D), jnp.float32),
            pltpu.SemaphoreType.DMA,
            pltpu.SemaphoreType.DMA,
        ],
    )


_prop64 = _make_prop(64)
_prop48 = _make_prop(48)


_DN = (((1,), (1,)), ((), ()))


def _dot(a, b):
    return lax.dot_general(a, b, _DN, preferred_element_type=jnp.float32)


def _kan_lhs(xb):
    u = 2.0 * xb + 5.0
    t = jnp.floor(u)
    f = u - t
    f2 = f * f
    f3 = f2 * f
    sixth = jnp.float32(1.0 / 6.0)
    v0 = f3 * sixth
    v1 = (-3.0 * f3 + 3.0 * f2 + 3.0 * f + 1.0) * sixth
    v2 = (3.0 * f3 - 6.0 * f2 + 4.0) * sixth
    omf = 1.0 - f
    v3 = omf * omf * omf * sixth
    cols = [jax.nn.silu(xb)]
    for c in range(7):
        fc = jnp.float32(c)
        cols.append(jnp.where(t == fc, v0, 0.0)
                    + jnp.where(t == fc + 1.0, v1, 0.0)
                    + jnp.where(t == fc + 2.0, v2, 0.0)
                    + jnp.where(t == fc + 3.0, v3, 0.0))
    return jnp.concatenate(cols, axis=1)


def _kan_rhs(wb, ws_t, scaler):
    return jnp.concatenate([wb] + [ws_t[c] * scaler for c in range(7)], axis=1)


def _dinv_of(cnt):
    return lax.rsqrt(cnt[0, :, 0:1] + cnt[1, :, 0:1] - 1.0)


def _kan1_body(x_ref, w1b_ref, w1s_ref, w1c_ref, w2b_ref, w2s_ref, w2c_ref,
               h_ref, hx_ref):
    lhs = _kan_lhs(x_ref[...])
    rhs = jnp.concatenate(
        [_kan_rhs(w1b_ref[...], w1s_ref, w1c_ref[...]),
         _kan_rhs(w2b_ref[...], w2s_ref, w2c_ref[...])], axis=0)
    out = _dot(lhs, rhs)
    h_ref[...] = out[:, :64]
    hx_ref[...] = out[:, 64:104]


def _scale_body(h_ref, cnt_ref, g_ref):
    g_ref[...] = h_ref[...] * _dinv_of(cnt_ref[...])


def _c1_body(s1_ref, g1_ref, cnt_ref, b1_ref, st_ref):
    i = pl.program_id(0)
    s1 = s1_ref[...]
    dinv = _dinv_of(cnt_ref[...])
    z = dinv * (s1[0] + s1[1] - g1_ref[...]) + b1_ref[...]

    @pl.when(i == 0)
    def _():
        st_ref[...] = jnp.zeros_like(st_ref)

    st_ref[...] = st_ref[...] + jnp.concatenate(
        [z.sum(0)[None, :], (z * z).sum(0)[None, :]], axis=0)


def _c2_body(s1_ref, g1_ref, cnt_ref, b1_ref, st_ref, gam_ref, bet_ref,
             hx_ref, w2b_ref, w2s_ref, w2c_ref, h_ref, g_ref):
    s1 = s1_ref[...]
    dinv = _dinv_of(cnt_ref[...])
    a1 = dinv * (s1[0] + s1[1] - g1_ref[...]) + b1_ref[...]
    st = st_ref[...]
    mean = st[0:1] * jnp.float32(1.0 / N)
    var = st[1:2] * jnp.float32(1.0 / N) - mean * mean
    hb = (a1 - mean) * lax.rsqrt(var + 1e-5) * gam_ref[...] + bet_ref[...]
    h = hx_ref[...] + _dot(_kan_lhs(hb),
                           _kan_rhs(w2b_ref[...], w2s_ref, w2c_ref[...]))
    h_ref[...] = h
    g_ref[...] = jnp.concatenate(
        [h * dinv, jnp.zeros((h.shape[0], 8), jnp.float32)], axis=1)


def _c3_body(s2_ref, g2_ref, cnt_ref, bo_ref, out_ref):
    s2 = s2_ref[...]
    dinv = _dinv_of(cnt_ref[...])
    out_ref[...] = (dinv * (s2[0] + s2[1] - g2_ref[...]))[:, :40] + bo_ref[...]


def _full(shape):
    return pl.BlockSpec(shape, lambda i: tuple(0 for _ in shape))


def _rows(d):
    return pl.BlockSpec((BLK, d), lambda i: (i, 0))


def _rows3(d):
    return pl.BlockSpec((NC, BLK, d), lambda i: (0, i, 0))


_CNT_SPEC = _rows3(HL)


def _padded(d):
    return jax.ShapeDtypeStruct((NP_, d), jnp.float32)


def _kan1_call(x, w1b, w1s_t, w1c, w2b, w2s_t, w2c):
    return pl.pallas_call(
        _kan1_body,
        grid=(GB,),
        in_specs=[_rows(128), _full((64, 128)), _full((7, 64, 128)),
                  _full((64, 128)), _full((40, 128)), _full((7, 40, 128)),
                  _full((40, 128))],
        out_specs=[_rows(64), _rows(40)],
        out_shape=[jax.ShapeDtypeStruct((N, 64), jnp.float32),
                   jax.ShapeDtypeStruct((N, 40), jnp.float32)],
    )(x, w1b, w1s_t, w1c, w2b, w2s_t, w2c)


def _scale_call(h1, hist):
    return pl.pallas_call(
        _scale_body,
        grid=(GB,),
        in_specs=[_rows(64), _CNT_SPEC],
        out_specs=_rows(64),
        out_shape=_padded(64),
    )(h1, hist)


def _c1_call(s1, g1, hist, b1):
    return pl.pallas_call(
        _c1_body,
        grid=(GB,),
        in_specs=[_rows3(64), _rows(64), _CNT_SPEC, _full((1, 64))],
        out_specs=_full((2, 64)),
        out_shape=jax.ShapeDtypeStruct((2, 64), jnp.float32),
    )(s1, g1, hist, b1)


def _c2_call(s1, g1, hist, b1, st, gam, bet, h2x, w2b, w2s_t, w2c):
    return pl.pallas_call(
        _c2_body,
        grid=(GB,),
        in_specs=[_rows3(64), _rows(64), _CNT_SPEC, _full((1, 64)),
                  _full((2, 64)), _full((1, 64)), _full((1, 64)), _rows(40),
                  _full((40, 64)), _full((7, 40, 64)), _full((40, 64))],
        out_specs=[_rows(40), _rows(48)],
        out_shape=[jax.ShapeDtypeStruct((N, 40), jnp.float32), _padded(48)],
    )(s1, g1, hist, b1, st, gam, bet, h2x, w2b, w2s_t, w2c)


def _c3_call(s2, g2, hist, bo):
    return pl.pallas_call(
        _c3_body,
        grid=(GB,),
        in_specs=[_rows3(48), _rows(48), _CNT_SPEC, _full((1, 40))],
        out_specs=_rows(40),
        out_shape=jax.ShapeDtypeStruct((N, 40), jnp.float32),
    )(s2, g2, hist, bo)


def kernel(x, edge_index, w1_base, w1_spline, w1_scaler, grid1, b1,
           bn_gamma, bn_beta, wo_base, wo_spline, wo_scaler, grido, bo):
    ei = edge_index.astype(jnp.int32).reshape(2, NC, NS, CH, K)
    w1_sp_t = jnp.transpose(w1_spline, (2, 0, 1))
    wo_sp_t = jnp.transpose(wo_spline, (2, 0, 1))

    hist = _hist(ei)
    h1, h2x = _kan1_call(x, w1_base, w1_sp_t, w1_scaler,
                         wo_base[:, :128], wo_sp_t[:, :, :128],
                         wo_scaler[:, :128])
    g1 = _scale_call(h1, hist)
    s1 = _prop64(ei, g1)
    st = _c1_call(s1, g1, hist, b1.reshape(1, 64))
    h2, g2 = _c2_call(s1, g1, hist, b1.reshape(1, 64), st,
                      bn_gamma.reshape(1, 64), bn_beta.reshape(1, 64), h2x,
                      wo_base[:, 128:], wo_sp_t[:, :, 128:], wo_scaler[:, 128:])
    s2 = _prop48(ei, g2)
    return _c3_call(s2, g2, hist, bo.reshape(1, 40))


# speedup vs baseline: 1.1312x; 1.0330x over previous
import functools

import jax
import jax.numpy as jnp
from jax import lax
from jax.experimental import pallas as pl
from jax.experimental.pallas import tpu as pltpu
from jax.experimental.pallas import tpu_sc as plsc

N = 10000
NP_ = 10240
E = 320000
NC = 2
NS = 16
K = 125
CH = E // (NC * NS * K)
NPT = NP_ // NS
HL = 16
BLK = 2000
GB = N // BLK

_SC_MESH = dict(core_axis_name="c", subcore_axis_name="s")
_SC_PARAMS = pltpu.CompilerParams(use_tc_tiling_on_sc=False)


@functools.partial(
    pl.kernel,
    mesh=plsc.VectorSubcoreMesh(**_SC_MESH),
    compiler_params=_SC_PARAMS,
    out_type=jax.ShapeDtypeStruct((NC, NP_, HL), jnp.float32),
    scratch_types=[
        pltpu.VMEM((CH, K), jnp.int32),
        pltpu.VMEM((128, HL), jnp.float32),
        pltpu.VMEM_SHARED((NP_, HL), jnp.float32),
    ],
)
def _hist(ei_hbm, out_hbm, idx_v, ones_v, acc):
    c = lax.axis_index("c")
    s = lax.axis_index("s")
    pltpu.sync_copy(ei_hbm.at[1, c, s], idx_v)

    def fill(i, carry):
        ones_v[i, :] = jnp.full((16,), 1.0, jnp.float32)
        return carry

    lax.fori_loop(0, 128, fill, 0)
    for b in range(NPT // 128):
        pltpu.sync_copy(ones_v, acc.at[pl.ds(s * NPT + b * 128, 128)])
    plsc.subcore_barrier()

    def body(j, carry):
        pltpu.sync_copy(ones_v.at[pl.ds(0, K)], acc.at[idx_v.at[j]], add=True)
        return carry

    lax.fori_loop(0, CH, body, 0)
    plsc.subcore_barrier()
    pltpu.sync_copy(acc.at[pl.ds(s * NPT, NPT)], out_hbm.at[c, pl.ds(s * NPT, NPT)])


def _make_prop(D):
    def prop_kernel(ei_hbm, g_hbm, out_hbm, srcv, dstv, rows_a, rows_b,
                    acc, gsem_a, gsem_b):
        c = lax.axis_index("c")
        s = lax.axis_index("s")
        pltpu.sync_copy(ei_hbm.at[0, c, s], srcv)
        pltpu.sync_copy(ei_hbm.at[1, c, s], dstv)
        pltpu.sync_copy(g_hbm.at[pl.ds(s * NPT, NPT)], acc.at[pl.ds(s * NPT, NPT)])
        plsc.subcore_barrier()

        pltpu.async_copy(g_hbm.at[srcv.at[0]], rows_a, gsem_a)

        def body(i, carry):
            ja = 2 * i
            jb = 2 * i + 1
            cp_b = pltpu.async_copy(g_hbm.at[srcv.at[jb]], rows_b, gsem_b)
            pltpu.make_async_copy(g_hbm.at[srcv.at[ja]], rows_a, gsem_a).wait()
            pltpu.sync_copy(rows_a, acc.at[dstv.at[ja]], add=True)

            @pl.when(jb + 1 < CH)
            def _():
                pltpu.async_copy(g_hbm.at[srcv.at[jb + 1]], rows_a, gsem_a)

            cp_b.wait()
            pltpu.sync_copy(rows_b, acc.at[dstv.at[jb]], add=True)
            return carry

        lax.fori_loop(0, CH // 2, body, 0)
        plsc.subcore_barrier()
        pltpu.sync_copy(acc.at[pl.ds(s * NPT, NPT)],
                        out_hbm.at[c, pl.ds(s * NPT, NPT)])

    return pl.kernel(
        prop_kernel,
        mesh=plsc.VectorSubcoreMesh(**_SC_MESH),
        compiler_params=_SC_PARAMS,
        out_type=jax.ShapeDtypeStruct((NC, NP_, D), jnp.float32),
        scratch_types=[
            pltpu.VMEM((CH, K), jnp.int32),
            pltpu.VMEM((CH, K), jnp.int32),
            pltpu.VMEM((K, D), jnp.float32),
            pltpu.VMEM((K, D), jnp.float32),
            pltpu.VMEM_SHARED((NP_, D), jnp.float32),
            pltpu.SemaphoreType.DMA,
            pltpu.SemaphoreType.DMA,
        ],
    )


_prop64 = _make_prop(64)
_prop40 = _make_prop(40)


_DN = (((1,), (1,)), ((), ()))


def _dot(a, b):
    return lax.dot_general(a, b, _DN, preferred_element_type=jnp.float32)


def _kan_lhs(xb):
    u = 2.0 * xb + 5.0
    t = jnp.floor(u)
    f = u - t
    f2 = f * f
    f3 = f2 * f
    sixth = jnp.float32(1.0 / 6.0)
    v0 = f3 * sixth
    v1 = (-3.0 * f3 + 3.0 * f2 + 3.0 * f + 1.0) * sixth
    v2 = (3.0 * f3 - 6.0 * f2 + 4.0) * sixth
    omf = 1.0 - f
    v3 = omf * omf * omf * sixth
    cols = [jax.nn.silu(xb)]
    for c in range(7):
        fc = jnp.float32(c)
        cols.append(jnp.where(t == fc, v0, 0.0)
                    + jnp.where(t == fc + 1.0, v1, 0.0)
                    + jnp.where(t == fc + 2.0, v2, 0.0)
                    + jnp.where(t == fc + 3.0, v3, 0.0))
    return jnp.concatenate(cols, axis=1)


def _kan_rhs(wb, ws_t, scaler):
    return jnp.concatenate([wb] + [ws_t[c] * scaler for c in range(7)], axis=1)


def _dinv_of(cnt):
    return lax.rsqrt(cnt[0, :, 0:1] + cnt[1, :, 0:1] - 1.0)


def _kan1_body(x_ref, w1b_ref, w1s_ref, w1c_ref, w2b_ref, w2s_ref, w2c_ref,
               h_ref, hx_ref):
    lhs = _kan_lhs(x_ref[...])
    rhs = jnp.concatenate(
        [_kan_rhs(w1b_ref[...], w1s_ref, w1c_ref[...]),
         _kan_rhs(w2b_ref[...], w2s_ref, w2c_ref[...])], axis=0)
    out = _dot(lhs, rhs)
    h_ref[...] = out[:, :64]
    hx_ref[...] = out[:, 64:104]


def _scale_body(h_ref, cnt_ref, g_ref):
    g_ref[...] = h_ref[...] * _dinv_of(cnt_ref[...])


def _c1_body(s1_ref, g1_ref, cnt_ref, b1_ref, st_ref):
    i = pl.program_id(0)
    s1 = s1_ref[...]
    dinv = _dinv_of(cnt_ref[...])
    z = dinv * (s1[0] + s1[1] - g1_ref[...]) + b1_ref[...]

    @pl.when(i == 0)
    def _():
        st_ref[...] = jnp.zeros_like(st_ref)

    st_ref[...] = st_ref[...] + jnp.concatenate(
        [z.sum(0)[None, :], (z * z).sum(0)[None, :]], axis=0)


def _c2_body(s1_ref, g1_ref, cnt_ref, b1_ref, st_ref, gam_ref, bet_ref,
             hx_ref, w2b_ref, w2s_ref, w2c_ref, h_ref, g_ref):
    s1 = s1_ref[...]
    dinv = _dinv_of(cnt_ref[...])
    a1 = dinv * (s1[0] + s1[1] - g1_ref[...]) + b1_ref[...]
    st = st_ref[...]
    mean = st[0:1] * jnp.float32(1.0 / N)
    var = st[1:2] * jnp.float32(1.0 / N) - mean * mean
    hb = (a1 - mean) * lax.rsqrt(var + 1e-5) * gam_ref[...] + bet_ref[...]
    h = hx_ref[...] + _dot(_kan_lhs(hb),
                           _kan_rhs(w2b_ref[...], w2s_ref, w2c_ref[...]))
    h_ref[...] = h
    g_ref[...] = h * dinv


def _c3_body(s2_ref, g2_ref, cnt_ref, bo_ref, out_ref):
    s2 = s2_ref[...]
    dinv = _dinv_of(cnt_ref[...])
    out_ref[...] = dinv * (s2[0] + s2[1] - g2_ref[...]) + bo_ref[...]


def _full(shape):
    return pl.BlockSpec(shape, lambda i: tuple(0 for _ in shape))


def _rows(d):
    return pl.BlockSpec((BLK, d), lambda i: (i, 0))


def _rows3(d):
    return pl.BlockSpec((NC, BLK, d), lambda i: (0, i, 0))


_CNT_SPEC = _rows3(HL)


def _padded(d):
    return jax.ShapeDtypeStruct((NP_, d), jnp.float32)


def _kan1_call(x, w1b, w1s_t, w1c, w2b, w2s_t, w2c):
    return pl.pallas_call(
        _kan1_body,
        grid=(GB,),
        in_specs=[_rows(128), _full((64, 128)), _full((7, 64, 128)),
                  _full((64, 128)), _full((40, 128)), _full((7, 40, 128)),
                  _full((40, 128))],
        out_specs=[_rows(64), _rows(40)],
        out_shape=[jax.ShapeDtypeStruct((N, 64), jnp.float32),
                   jax.ShapeDtypeStruct((N, 40), jnp.float32)],
    )(x, w1b, w1s_t, w1c, w2b, w2s_t, w2c)


def _scale_call(h1, hist):
    return pl.pallas_call(
        _scale_body,
        grid=(GB,),
        in_specs=[_rows(64), _CNT_SPEC],
        out_specs=_rows(64),
        out_shape=_padded(64),
    )(h1, hist)


def _c1_call(s1, g1, hist, b1):
    return pl.pallas_call(
        _c1_body,
        grid=(GB,),
        in_specs=[_rows3(64), _rows(64), _CNT_SPEC, _full((1, 64))],
        out_specs=_full((2, 64)),
        out_shape=jax.ShapeDtypeStruct((2, 64), jnp.float32),
    )(s1, g1, hist, b1)


def _c2_call(s1, g1, hist, b1, st, gam, bet, h2x, w2b, w2s_t, w2c):
    return pl.pallas_call(
        _c2_body,
        grid=(GB,),
        in_specs=[_rows3(64), _rows(64), _CNT_SPEC, _full((1, 64)),
                  _full((2, 64)), _full((1, 64)), _full((1, 64)), _rows(40),
                  _full((40, 64)), _full((7, 40, 64)), _full((40, 64))],
        out_specs=[_rows(40), _rows(40)],
        out_shape=[jax.ShapeDtypeStruct((N, 40), jnp.float32), _padded(40)],
    )(s1, g1, hist, b1, st, gam, bet, h2x, w2b, w2s_t, w2c)


def _c3_call(s2, g2, hist, bo):
    return pl.pallas_call(
        _c3_body,
        grid=(GB,),
        in_specs=[_rows3(40), _rows(40), _CNT_SPEC, _full((1, 40))],
        out_specs=_rows(40),
        out_shape=jax.ShapeDtypeStruct((N, 40), jnp.float32),
    )(s2, g2, hist, bo)


def kernel(x, edge_index, w1_base, w1_spline, w1_scaler, grid1, b1,
           bn_gamma, bn_beta, wo_base, wo_spline, wo_scaler, grido, bo):
    ei = edge_index.astype(jnp.int32).reshape(2, NC, NS, CH, K)
    w1_sp_t = jnp.transpose(w1_spline, (2, 0, 1))
    wo_sp_t = jnp.transpose(wo_spline, (2, 0, 1))

    hist = _hist(ei)
    h1, h2x = _kan1_call(x, w1_base, w1_sp_t, w1_scaler,
                         wo_base[:, :128], wo_sp_t[:, :, :128],
                         wo_scaler[:, :128])
    g1 = _scale_call(h1, hist)
    s1 = _prop64(ei, g1)
    st = _c1_call(s1, g1, hist, b1.reshape(1, 64))
    h2, g2 = _c2_call(s1, g1, hist, b1.reshape(1, 64), st,
                      bn_gamma.reshape(1, 64), bn_beta.reshape(1, 64), h2x,
                      wo_base[:, 128:], wo_sp_t[:, :, 128:], wo_scaler[:, 128:])
    s2 = _prop40(ei, g2)
    return _c3_call(s2, g2, hist, bo.reshape(1, 40))


# speedup vs baseline: 1.1336x; 1.0022x over previous
import functools

import jax
import jax.numpy as jnp
from jax import lax
from jax.experimental import pallas as pl
from jax.experimental.pallas import tpu as pltpu
from jax.experimental.pallas import tpu_sc as plsc

N = 10000
NP_ = 10240
E = 320000
NC = 2
NS = 16
K = 125
CH = E // (NC * NS * K)
NPT = NP_ // NS
HL = 16
BLK = 2000
GB = N // BLK

_SC_MESH = dict(core_axis_name="c", subcore_axis_name="s")
_SC_PARAMS = pltpu.CompilerParams(use_tc_tiling_on_sc=False)


@functools.partial(
    pl.kernel,
    mesh=plsc.VectorSubcoreMesh(**_SC_MESH),
    compiler_params=_SC_PARAMS,
    out_type=jax.ShapeDtypeStruct((NC, NP_, HL), jnp.float32),
    scratch_types=[
        pltpu.VMEM((CH, K), jnp.int32),
        pltpu.VMEM((128, HL), jnp.float32),
        pltpu.VMEM_SHARED((NP_, HL), jnp.float32),
    ],
)
def _hist(ei_hbm, out_hbm, idx_v, ones_v, acc):
    c = lax.axis_index("c")
    s = lax.axis_index("s")
    pltpu.sync_copy(ei_hbm.at[1, c, s], idx_v)

    def fill(i, carry):
        ones_v[i, :] = jnp.full((16,), 1.0, jnp.float32)
        return carry

    lax.fori_loop(0, 128, fill, 0)
    for b in range(NPT // 128):
        pltpu.sync_copy(ones_v, acc.at[pl.ds(s * NPT + b * 128, 128)])
    plsc.subcore_barrier()

    def body(j, carry):
        pltpu.sync_copy(ones_v.at[pl.ds(0, K)], acc.at[idx_v.at[j]], add=True)
        return carry

    lax.fori_loop(0, CH, body, 0)
    plsc.subcore_barrier()
    pltpu.sync_copy(acc.at[pl.ds(s * NPT, NPT)], out_hbm.at[c, pl.ds(s * NPT, NPT)])


def _make_prop(D):
    def prop_kernel(ei_hbm, g_hbm, out_hbm, srcv, dstv, rows_a, rows_b,
                    acc, gsem_a, gsem_b):
        c = lax.axis_index("c")
        s = lax.axis_index("s")
        pltpu.sync_copy(ei_hbm.at[0, c, s], srcv)
        pltpu.sync_copy(ei_hbm.at[1, c, s], dstv)
        pltpu.sync_copy(g_hbm.at[pl.ds(s * NPT, NPT)], acc.at[pl.ds(s * NPT, NPT)])
        plsc.subcore_barrier()

        pltpu.async_copy(g_hbm.at[srcv.at[0]], rows_a, gsem_a)

        def body(i, carry):
            ja = 2 * i
            jb = 2 * i + 1
            cp_b = pltpu.async_copy(g_hbm.at[srcv.at[jb]], rows_b, gsem_b)
            pltpu.make_async_copy(g_hbm.at[srcv.at[ja]], rows_a, gsem_a).wait()
            pltpu.sync_copy(rows_a, acc.at[dstv.at[ja]], add=True)

            @pl.when(jb + 1 < CH)
            def _():
                pltpu.async_copy(g_hbm.at[srcv.at[jb + 1]], rows_a, gsem_a)

            cp_b.wait()
            pltpu.sync_copy(rows_b, acc.at[dstv.at[jb]], add=True)
            return carry

        lax.fori_loop(0, CH // 2, body, 0)
        plsc.subcore_barrier()
        pltpu.sync_copy(acc.at[pl.ds(s * NPT, NPT)],
                        out_hbm.at[c, pl.ds(s * NPT, NPT)])

    return pl.kernel(
        prop_kernel,
        mesh=plsc.VectorSubcoreMesh(**_SC_MESH),
        compiler_params=_SC_PARAMS,
        out_type=jax.ShapeDtypeStruct((NC, NP_, D), jnp.float32),
        scratch_types=[
            pltpu.VMEM((CH, K), jnp.int32),
            pltpu.VMEM((CH, K), jnp.int32),
            pltpu.VMEM((K, D), jnp.float32),
            pltpu.VMEM((K, D), jnp.float32),
            pltpu.VMEM_SHARED((NP_, D), jnp.float32),
            pltpu.SemaphoreType.DMA,
            pltpu.SemaphoreType.DMA,
        ],
    )


_prop64 = _make_prop(64)
_prop40 = _make_prop(40)


_DN = (((1,), (1,)), ((), ()))


def _dot(a, b):
    return lax.dot_general(a, b, _DN, preferred_element_type=jnp.float32)


def _kan_lhs(xb):
    u = 2.0 * xb + 5.0
    t = jnp.floor(u)
    f = u - t
    f2 = f * f
    f3 = f2 * f
    sixth = jnp.float32(1.0 / 6.0)
    v0 = f3 * sixth
    v1 = (-3.0 * f3 + 3.0 * f2 + 3.0 * f + 1.0) * sixth
    v2 = (3.0 * f3 - 6.0 * f2 + 4.0) * sixth
    omf = 1.0 - f
    v3 = omf * omf * omf * sixth
    cols = [jax.nn.silu(xb)]
    for c in range(7):
        fc = jnp.float32(c)
        cols.append(jnp.where(t == fc, v0, 0.0)
                    + jnp.where(t == fc + 1.0, v1, 0.0)
                    + jnp.where(t == fc + 2.0, v2, 0.0)
                    + jnp.where(t == fc + 3.0, v3, 0.0))
    return jnp.concatenate(cols, axis=1)


def _kan_rhs(wb, ws_t, scaler):
    return jnp.concatenate([wb] + [ws_t[c] * scaler for c in range(7)], axis=1)


def _dinv_of(cnt):
    return lax.rsqrt(cnt[0, :, 0:1] + cnt[1, :, 0:1] - 1.0)


def _kan1_body(x_ref, w1b_ref, w1s_ref, w1c_ref, w2b_ref, w2s_ref, w2c_ref,
               h_ref, hx_ref):
    lhs = _kan_lhs(x_ref[...])
    rhs = jnp.concatenate(
        [_kan_rhs(w1b_ref[...], w1s_ref, w1c_ref[...]),
         _kan_rhs(w2b_ref[...], w2s_ref, w2c_ref[...])], axis=0)
    out = _dot(lhs, rhs)
    h_ref[...] = out[:, :64]
    hx_ref[...] = out[:, 64:104]


def _scale_body(h_ref, cnt_ref, g_ref, d_ref):
    dinv = _dinv_of(cnt_ref[...])
    g_ref[...] = h_ref[...] * dinv
    d_ref[...] = jnp.broadcast_to(dinv, d_ref.shape)


def _c1_body(s1_ref, g1_ref, d_ref, b1_ref, st_ref):
    i = pl.program_id(0)
    s1 = s1_ref[...]
    z = d_ref[...] * (s1[0] + s1[1] - g1_ref[...]) + b1_ref[...]

    @pl.when(i == 0)
    def _():
        st_ref[...] = jnp.zeros_like(st_ref)

    st_ref[...] = st_ref[...] + jnp.concatenate(
        [z.sum(0)[None, :], (z * z).sum(0)[None, :]], axis=0)


def _c2_body(s1_ref, g1_ref, d_ref, b1_ref, st_ref, gam_ref, bet_ref,
             hx_ref, w2b_ref, w2s_ref, w2c_ref, h_ref, g_ref):
    s1 = s1_ref[...]
    dinv = d_ref[...]
    a1 = dinv * (s1[0] + s1[1] - g1_ref[...]) + b1_ref[...]
    st = st_ref[...]
    mean = st[0:1] * jnp.float32(1.0 / N)
    var = st[1:2] * jnp.float32(1.0 / N) - mean * mean
    hb = (a1 - mean) * lax.rsqrt(var + 1e-5) * gam_ref[...] + bet_ref[...]
    h = hx_ref[...] + _dot(_kan_lhs(hb),
                           _kan_rhs(w2b_ref[...], w2s_ref, w2c_ref[...]))
    h_ref[...] = h
    g_ref[...] = h * dinv[:, :40]


def _c3_body(s2_ref, g2_ref, d_ref, bo_ref, out_ref):
    s2 = s2_ref[...]
    out_ref[...] = (d_ref[...][:, :40] * (s2[0] + s2[1] - g2_ref[...])
                    + bo_ref[...])


def _full(shape):
    return pl.BlockSpec(shape, lambda i: tuple(0 for _ in shape))


def _rows(d):
    return pl.BlockSpec((BLK, d), lambda i: (i, 0))


def _rows3(d):
    return pl.BlockSpec((NC, BLK, d), lambda i: (0, i, 0))


_CNT_SPEC = _rows3(HL)


def _padded(d):
    return jax.ShapeDtypeStruct((NP_, d), jnp.float32)


def _kan1_call(x, w1b, w1s_t, w1c, w2b, w2s_t, w2c):
    return pl.pallas_call(
        _kan1_body,
        grid=(GB,),
        in_specs=[_rows(128), _full((64, 128)), _full((7, 64, 128)),
                  _full((64, 128)), _full((40, 128)), _full((7, 40, 128)),
                  _full((40, 128))],
        out_specs=[_rows(64), _rows(40)],
        out_shape=[jax.ShapeDtypeStruct((N, 64), jnp.float32),
                   jax.ShapeDtypeStruct((N, 40), jnp.float32)],
    )(x, w1b, w1s_t, w1c, w2b, w2s_t, w2c)


def _scale_call(h1, hist):
    return pl.pallas_call(
        _scale_body,
        grid=(GB,),
        in_specs=[_rows(64), _CNT_SPEC],
        out_specs=[_rows(64), _rows(64)],
        out_shape=[_padded(64), _padded(64)],
    )(h1, hist)


def _c1_call(s1, g1, hist, b1):
    return pl.pallas_call(
        _c1_body,
        grid=(GB,),
        in_specs=[_rows3(64), _rows(64), _rows(64), _full((1, 64))],
        out_specs=_full((2, 64)),
        out_shape=jax.ShapeDtypeStruct((2, 64), jnp.float32),
    )(s1, g1, hist, b1)


def _c2_call(s1, g1, hist, b1, st, gam, bet, h2x, w2b, w2s_t, w2c):
    return pl.pallas_call(
        _c2_body,
        grid=(GB,),
        in_specs=[_rows3(64), _rows(64), _rows(64), _full((1, 64)),
                  _full((2, 64)), _full((1, 64)), _full((1, 64)), _rows(40),
                  _full((40, 64)), _full((7, 40, 64)), _full((40, 64))],
        out_specs=[_rows(40), _rows(40)],
        out_shape=[jax.ShapeDtypeStruct((N, 40), jnp.float32), _padded(40)],
    )(s1, g1, hist, b1, st, gam, bet, h2x, w2b, w2s_t, w2c)


def _c3_call(s2, g2, hist, bo):
    return pl.pallas_call(
        _c3_body,
        grid=(GB,),
        in_specs=[_rows3(40), _rows(40), _rows(64), _full((1, 40))],
        out_specs=_rows(40),
        out_shape=jax.ShapeDtypeStruct((N, 40), jnp.float32),
    )(s2, g2, hist, bo)


def kernel(x, edge_index, w1_base, w1_spline, w1_scaler, grid1, b1,
           bn_gamma, bn_beta, wo_base, wo_spline, wo_scaler, grido, bo):
    ei = edge_index.astype(jnp.int32).reshape(2, NC, NS, CH, K)
    w1_sp_t = jnp.transpose(w1_spline, (2, 0, 1))
    wo_sp_t = jnp.transpose(wo_spline, (2, 0, 1))

    hist = _hist(ei)
    h1, h2x = _kan1_call(x, w1_base, w1_sp_t, w1_scaler,
                         wo_base[:, :128], wo_sp_t[:, :, :128],
                         wo_scaler[:, :128])
    g1, dinv64 = _scale_call(h1, hist)
    s1 = _prop64(ei, g1)
    st = _c1_call(s1, g1, dinv64, b1.reshape(1, 64))
    h2, g2 = _c2_call(s1, g1, dinv64, b1.reshape(1, 64), st,
                      bn_gamma.reshape(1, 64), bn_beta.reshape(1, 64), h2x,
                      wo_base[:, 128:], wo_sp_t[:, :, 128:], wo_scaler[:, 128:])
    s2 = _prop40(ei, g2)
    return _c3_call(s2, g2, dinv64, bo.reshape(1, 40))


# speedup vs baseline: 1.3043x; 1.1505x over previous
import functools

import jax
import jax.numpy as jnp
from jax import lax
from jax.experimental import pallas as pl
from jax.experimental.pallas import tpu as pltpu
from jax.experimental.pallas import tpu_sc as plsc

N = 10000
NP_ = 10240
E = 320000
NC = 2
NS = 16
K = 125
CH = E // (NC * NS * K)
NPT = NP_ // NS
HL = 16
BLK = 2000
GB = N // BLK

_SC_MESH = dict(core_axis_name="c", subcore_axis_name="s")
_SC_PARAMS = pltpu.CompilerParams(use_tc_tiling_on_sc=False)


@functools.partial(
    pl.kernel,
    mesh=plsc.VectorSubcoreMesh(**_SC_MESH),
    compiler_params=_SC_PARAMS,
    out_type=jax.ShapeDtypeStruct((NC, NP_, HL), jnp.float32),
    scratch_types=[
        pltpu.VMEM((CH, K), jnp.int32),
        pltpu.VMEM((128, HL), jnp.float32),
        pltpu.VMEM_SHARED((NP_, HL), jnp.float32),
    ],
)
def _hist(ei_hbm, out_hbm, idx_v, ones_v, acc):
    c = lax.axis_index("c")
    s = lax.axis_index("s")
    pltpu.sync_copy(ei_hbm.at[1, c, s], idx_v)

    def fill(i, carry):
        ones_v[i, :] = jnp.full((16,), 1.0, jnp.float32)
        return carry

    lax.fori_loop(0, 128, fill, 0)
    for b in range(NPT // 128):
        pltpu.sync_copy(ones_v, acc.at[pl.ds(s * NPT + b * 128, 128)])
    plsc.subcore_barrier()

    def body(j, carry):
        pltpu.sync_copy(ones_v.at[pl.ds(0, K)], acc.at[idx_v.at[j]], add=True)
        return carry

    lax.fori_loop(0, CH, body, 0)
    plsc.subcore_barrier()
    pltpu.sync_copy(acc.at[pl.ds(s * NPT, NPT)], out_hbm.at[c, pl.ds(s * NPT, NPT)])


def _make_prop(D):
    NB = 4

    def prop_kernel(ei_hbm, g_hbm, out_hbm, srcv, dstv, rows, acc, *sems):
        c = lax.axis_index("c")
        s = lax.axis_index("s")
        pltpu.sync_copy(ei_hbm.at[0, c, s], srcv)
        pltpu.sync_copy(ei_hbm.at[1, c, s], dstv)
        pltpu.sync_copy(g_hbm.at[pl.ds(s * NPT, NPT)], acc.at[pl.ds(s * NPT, NPT)])
        plsc.subcore_barrier()

        for u in range(NB):
            pltpu.async_copy(g_hbm.at[srcv.at[u]], rows[u], sems[u])

        def body(i, carry):
            j0 = NB * i
            for u in range(NB):
                pltpu.make_async_copy(g_hbm.at[srcv.at[j0 + u]],
                                      rows[u], sems[u]).wait()
                pltpu.sync_copy(rows[u], acc.at[dstv.at[j0 + u]], add=True)
                pltpu.async_copy(g_hbm.at[srcv.at[j0 + u + NB]],
                                 rows[u], sems[u])
            return carry

        lax.fori_loop(0, CH // NB - 1, body, 0)
        j0 = CH - NB
        for u in range(NB):
            pltpu.make_async_copy(g_hbm.at[srcv.at[j0 + u]],
                                  rows[u], sems[u]).wait()
            pltpu.sync_copy(rows[u], acc.at[dstv.at[j0 + u]], add=True)
        plsc.subcore_barrier()
        pltpu.sync_copy(acc.at[pl.ds(s * NPT, NPT)],
                        out_hbm.at[c, pl.ds(s * NPT, NPT)])

    return pl.kernel(
        prop_kernel,
        mesh=plsc.VectorSubcoreMesh(**_SC_MESH),
        compiler_params=_SC_PARAMS,
        out_type=jax.ShapeDtypeStruct((NC, NP_, D), jnp.float32),
        scratch_types=[
            pltpu.VMEM((CH, K), jnp.int32),
            pltpu.VMEM((CH, K), jnp.int32),
            [pltpu.VMEM((K, D), jnp.float32)] * NB,
            pltpu.VMEM_SHARED((NP_, D), jnp.float32),
        ] + [pltpu.SemaphoreType.DMA] * NB,
    )


_prop64 = _make_prop(64)
_prop40 = _make_prop(40)


_DN = (((1,), (1,)), ((), ()))


def _dot(a, b):
    return lax.dot_general(a, b, _DN, preferred_element_type=jnp.float32)


def _kan_lhs(xb):
    u = 2.0 * xb + 5.0
    t = jnp.floor(u)
    f = u - t
    f2 = f * f
    f3 = f2 * f
    sixth = jnp.float32(1.0 / 6.0)
    v0 = f3 * sixth
    v1 = (-3.0 * f3 + 3.0 * f2 + 3.0 * f + 1.0) * sixth
    v2 = (3.0 * f3 - 6.0 * f2 + 4.0) * sixth
    omf = 1.0 - f
    v3 = omf * omf * omf * sixth
    cols = [jax.nn.silu(xb)]
    for c in range(7):
        fc = jnp.float32(c)
        cols.append(jnp.where(t == fc, v0, 0.0)
                    + jnp.where(t == fc + 1.0, v1, 0.0)
                    + jnp.where(t == fc + 2.0, v2, 0.0)
                    + jnp.where(t == fc + 3.0, v3, 0.0))
    return jnp.concatenate(cols, axis=1)


def _kan_rhs(wb, ws_t, scaler):
    return jnp.concatenate([wb] + [ws_t[c] * scaler for c in range(7)], axis=1)


def _dinv_of(cnt):
    return lax.rsqrt(cnt[0, :, 0:1] + cnt[1, :, 0:1] - 1.0)


def _kan1_body(x_ref, w1b_ref, w1s_ref, w1c_ref, w2b_ref, w2s_ref, w2c_ref,
               h_ref, hx_ref):
    lhs = _kan_lhs(x_ref[...])
    rhs = jnp.concatenate(
        [_kan_rhs(w1b_ref[...], w1s_ref, w1c_ref[...]),
         _kan_rhs(w2b_ref[...], w2s_ref, w2c_ref[...])], axis=0)
    out = _dot(lhs, rhs)
    h_ref[...] = out[:, :64]
    hx_ref[...] = out[:, 64:104]


def _scale_body(h_ref, cnt_ref, g_ref, d_ref):
    dinv = _dinv_of(cnt_ref[...])
    g_ref[...] = h_ref[...] * dinv
    d_ref[...] = jnp.broadcast_to(dinv, d_ref.shape)


def _c1_body(s1_ref, g1_ref, d_ref, b1_ref, st_ref):
    i = pl.program_id(0)
    s1 = s1_ref[...]
    z = d_ref[...] * (s1[0] + s1[1] - g1_ref[...]) + b1_ref[...]

    @pl.when(i == 0)
    def _():
        st_ref[...] = jnp.zeros_like(st_ref)

    st_ref[...] = st_ref[...] + jnp.concatenate(
        [z.sum(0)[None, :], (z * z).sum(0)[None, :]], axis=0)


def _c2_body(s1_ref, g1_ref, d_ref, b1_ref, st_ref, gam_ref, bet_ref,
             hx_ref, w2b_ref, w2s_ref, w2c_ref, h_ref, g_ref):
    s1 = s1_ref[...]
    dinv = d_ref[...]
    a1 = dinv * (s1[0] + s1[1] - g1_ref[...]) + b1_ref[...]
    st = st_ref[...]
    mean = st[0:1] * jnp.float32(1.0 / N)
    var = st[1:2] * jnp.float32(1.0 / N) - mean * mean
    hb = (a1 - mean) * lax.rsqrt(var + 1e-5) * gam_ref[...] + bet_ref[...]
    h = hx_ref[...] + _dot(_kan_lhs(hb),
                           _kan_rhs(w2b_ref[...], w2s_ref, w2c_ref[...]))
    h_ref[...] = h
    g_ref[...] = h * dinv[:, :40]


def _c3_body(s2_ref, g2_ref, d_ref, bo_ref, out_ref):
    s2 = s2_ref[...]
    out_ref[...] = (d_ref[...][:, :40] * (s2[0] + s2[1] - g2_ref[...])
                    + bo_ref[...])


def _full(shape):
    return pl.BlockSpec(shape, lambda i: tuple(0 for _ in shape))


def _rows(d):
    return pl.BlockSpec((BLK, d), lambda i: (i, 0))


def _rows3(d):
    return pl.BlockSpec((NC, BLK, d), lambda i: (0, i, 0))


_CNT_SPEC = _rows3(HL)


def _padded(d):
    return jax.ShapeDtypeStruct((NP_, d), jnp.float32)


def _kan1_call(x, w1b, w1s_t, w1c, w2b, w2s_t, w2c):
    return pl.pallas_call(
        _kan1_body,
        grid=(GB,),
        in_specs=[_rows(128), _full((64, 128)), _full((7, 64, 128)),
                  _full((64, 128)), _full((40, 128)), _full((7, 40, 128)),
                  _full((40, 128))],
        out_specs=[_rows(64), _rows(40)],
        out_shape=[jax.ShapeDtypeStruct((N, 64), jnp.float32),
                   jax.ShapeDtypeStruct((N, 40), jnp.float32)],
    )(x, w1b, w1s_t, w1c, w2b, w2s_t, w2c)


def _scale_call(h1, hist):
    return pl.pallas_call(
        _scale_body,
        grid=(GB,),
        in_specs=[_rows(64), _CNT_SPEC],
        out_specs=[_rows(64), _rows(64)],
        out_shape=[_padded(64), _padded(64)],
    )(h1, hist)


def _c1_call(s1, g1, hist, b1):
    return pl.pallas_call(
        _c1_body,
        grid=(GB,),
        in_specs=[_rows3(64), _rows(64), _rows(64), _full((1, 64))],
        out_specs=_full((2, 64)),
        out_shape=jax.ShapeDtypeStruct((2, 64), jnp.float32),
    )(s1, g1, hist, b1)


def _c2_call(s1, g1, hist, b1, st, gam, bet, h2x, w2b, w2s_t, w2c):
    return pl.pallas_call(
        _c2_body,
        grid=(GB,),
        in_specs=[_rows3(64), _rows(64), _rows(64), _full((1, 64)),
                  _full((2, 64)), _full((1, 64)), _full((1, 64)), _rows(40),
                  _full((40, 64)), _full((7, 40, 64)), _full((40, 64))],
        out_specs=[_rows(40), _rows(40)],
        out_shape=[jax.ShapeDtypeStruct((N, 40), jnp.float32), _padded(40)],
    )(s1, g1, hist, b1, st, gam, bet, h2x, w2b, w2s_t, w2c)


def _c3_call(s2, g2, hist, bo):
    return pl.pallas_call(
        _c3_body,
        grid=(GB,),
        in_specs=[_rows3(40), _rows(40), _rows(64), _full((1, 40))],
        out_specs=_rows(40),
        out_shape=jax.ShapeDtypeStruct((N, 40), jnp.float32),
    )(s2, g2, hist, bo)


def kernel(x, edge_index, w1_base, w1_spline, w1_scaler, grid1, b1,
           bn_gamma, bn_beta, wo_base, wo_spline, wo_scaler, grido, bo):
    ei = edge_index.astype(jnp.int32).reshape(2, NC, NS, CH, K)
    w1_sp_t = jnp.transpose(w1_spline, (2, 0, 1))
    wo_sp_t = jnp.transpose(wo_spline, (2, 0, 1))

    hist = _hist(ei)
    h1, h2x = _kan1_call(x, w1_base, w1_sp_t, w1_scaler,
                         wo_base[:, :128], wo_sp_t[:, :, :128],
                         wo_scaler[:, :128])
    g1, dinv64 = _scale_call(h1, hist)
    s1 = _prop64(ei, g1)
    st = _c1_call(s1, g1, dinv64, b1.reshape(1, 64))
    h2, g2 = _c2_call(s1, g1, dinv64, b1.reshape(1, 64), st,
                      bn_gamma.reshape(1, 64), bn_beta.reshape(1, 64), h2x,
                      wo_base[:, 128:], wo_sp_t[:, :, 128:], wo_scaler[:, 128:])
    s2 = _prop40(ei, g2)
    return _c3_call(s2, g2, dinv64, bo.reshape(1, 40))


# speedup vs baseline: 1.3137x; 1.0072x over previous
import functools

import jax
import jax.numpy as jnp
from jax import lax
from jax.experimental import pallas as pl
from jax.experimental.pallas import tpu as pltpu
from jax.experimental.pallas import tpu_sc as plsc

N = 10000
NP_ = 10240
E = 320000
NC = 2
NS = 16
K = 125
CH = E // (NC * NS * K)
NPT = NP_ // NS
HL = 16
BLK = 2000
GB = N // BLK

_SC_MESH = dict(core_axis_name="c", subcore_axis_name="s")
_SC_PARAMS = pltpu.CompilerParams(use_tc_tiling_on_sc=False)


@functools.partial(
    pl.kernel,
    mesh=plsc.VectorSubcoreMesh(**_SC_MESH),
    compiler_params=_SC_PARAMS,
    out_type=jax.ShapeDtypeStruct((NC, NP_, HL), jnp.float32),
    scratch_types=[
        pltpu.VMEM((CH, K), jnp.int32),
        pltpu.VMEM((128, HL), jnp.float32),
        pltpu.VMEM_SHARED((NP_, HL), jnp.float32),
    ],
)
def _hist(ei_hbm, out_hbm, idx_v, ones_v, acc):
    c = lax.axis_index("c")
    s = lax.axis_index("s")
    pltpu.sync_copy(ei_hbm.at[1, c, s], idx_v)

    def fill(i, carry):
        ones_v[i, :] = jnp.full((16,), 1.0, jnp.float32)
        return carry

    lax.fori_loop(0, 128, fill, 0)
    for b in range(NPT // 128):
        pltpu.sync_copy(ones_v, acc.at[pl.ds(s * NPT + b * 128, 128)])
    plsc.subcore_barrier()

    def body(j, carry):
        pltpu.sync_copy(ones_v.at[pl.ds(0, K)], acc.at[idx_v.at[j]], add=True)
        return carry

    lax.fori_loop(0, CH, body, 0)
    plsc.subcore_barrier()
    pltpu.sync_copy(acc.at[pl.ds(s * NPT, NPT)], out_hbm.at[c, pl.ds(s * NPT, NPT)])


def _make_prop(D):
    NB = 8

    def prop_kernel(ei_hbm, g_hbm, out_hbm, srcv, dstv, rows, acc, *sems):
        c = lax.axis_index("c")
        s = lax.axis_index("s")
        pltpu.sync_copy(ei_hbm.at[0, c, s], srcv)
        pltpu.sync_copy(ei_hbm.at[1, c, s], dstv)
        pltpu.sync_copy(g_hbm.at[pl.ds(s * NPT, NPT)], acc.at[pl.ds(s * NPT, NPT)])
        plsc.subcore_barrier()

        for u in range(NB):
            pltpu.async_copy(g_hbm.at[srcv.at[u]], rows[u], sems[u])

        def body(i, carry):
            j0 = NB * i
            for u in range(NB):
                pltpu.make_async_copy(g_hbm.at[srcv.at[j0 + u]],
                                      rows[u], sems[u]).wait()
                pltpu.sync_copy(rows[u], acc.at[dstv.at[j0 + u]], add=True)
                pltpu.async_copy(g_hbm.at[srcv.at[j0 + u + NB]],
                                 rows[u], sems[u])
            return carry

        lax.fori_loop(0, CH // NB - 1, body, 0)
        j0 = CH - NB
        for u in range(NB):
            pltpu.make_async_copy(g_hbm.at[srcv.at[j0 + u]],
                                  rows[u], sems[u]).wait()
            pltpu.sync_copy(rows[u], acc.at[dstv.at[j0 + u]], add=True)
        plsc.subcore_barrier()
        pltpu.sync_copy(acc.at[pl.ds(s * NPT, NPT)],
                        out_hbm.at[c, pl.ds(s * NPT, NPT)])

    return pl.kernel(
        prop_kernel,
        mesh=plsc.VectorSubcoreMesh(**_SC_MESH),
        compiler_params=_SC_PARAMS,
        out_type=jax.ShapeDtypeStruct((NC, NP_, D), jnp.float32),
        scratch_types=[
            pltpu.VMEM((CH, K), jnp.int32),
            pltpu.VMEM((CH, K), jnp.int32),
            [pltpu.VMEM((K, D), jnp.float32)] * NB,
            pltpu.VMEM_SHARED((NP_, D), jnp.float32),
        ] + [pltpu.SemaphoreType.DMA] * NB,
    )


_prop64 = _make_prop(64)
_prop40 = _make_prop(40)


_DN = (((1,), (1,)), ((), ()))


def _dot(a, b):
    return lax.dot_general(a, b, _DN, preferred_element_type=jnp.float32)


def _kan_lhs(xb):
    u = 2.0 * xb + 5.0
    t = jnp.floor(u)
    f = u - t
    f2 = f * f
    f3 = f2 * f
    sixth = jnp.float32(1.0 / 6.0)
    v0 = f3 * sixth
    v1 = (-3.0 * f3 + 3.0 * f2 + 3.0 * f + 1.0) * sixth
    v2 = (3.0 * f3 - 6.0 * f2 + 4.0) * sixth
    omf = 1.0 - f
    v3 = omf * omf * omf * sixth
    cols = [jax.nn.silu(xb)]
    for c in range(7):
        fc = jnp.float32(c)
        cols.append(jnp.where(t == fc, v0, 0.0)
                    + jnp.where(t == fc + 1.0, v1, 0.0)
                    + jnp.where(t == fc + 2.0, v2, 0.0)
                    + jnp.where(t == fc + 3.0, v3, 0.0))
    return jnp.concatenate(cols, axis=1)


def _kan_rhs(wb, ws_t, scaler):
    return jnp.concatenate([wb] + [ws_t[c] * scaler for c in range(7)], axis=1)


def _dinv_of(cnt):
    return lax.rsqrt(cnt[0, :, 0:1] + cnt[1, :, 0:1] - 1.0)


def _kan1_body(x_ref, w1b_ref, w1s_ref, w1c_ref, w2b_ref, w2s_ref, w2c_ref,
               h_ref, hx_ref):
    lhs = _kan_lhs(x_ref[...])
    rhs = jnp.concatenate(
        [_kan_rhs(w1b_ref[...], w1s_ref, w1c_ref[...]),
         _kan_rhs(w2b_ref[...], w2s_ref, w2c_ref[...])], axis=0)
    out = _dot(lhs, rhs)
    h_ref[...] = out[:, :64]
    hx_ref[...] = out[:, 64:104]


def _scale_body(h_ref, cnt_ref, g_ref, d_ref):
    dinv = _dinv_of(cnt_ref[...])
    g_ref[...] = h_ref[...] * dinv
    d_ref[...] = jnp.broadcast_to(dinv, d_ref.shape)


def _c1_body(s1_ref, g1_ref, d_ref, b1_ref, st_ref):
    i = pl.program_id(0)
    s1 = s1_ref[...]
    z = d_ref[...] * (s1[0] + s1[1] - g1_ref[...]) + b1_ref[...]

    @pl.when(i == 0)
    def _():
        st_ref[...] = jnp.zeros_like(st_ref)

    st_ref[...] = st_ref[...] + jnp.concatenate(
        [z.sum(0)[None, :], (z * z).sum(0)[None, :]], axis=0)


def _c2_body(s1_ref, g1_ref, d_ref, b1_ref, st_ref, gam_ref, bet_ref,
             hx_ref, w2b_ref, w2s_ref, w2c_ref, h_ref, g_ref):
    s1 = s1_ref[...]
    dinv = d_ref[...]
    a1 = dinv * (s1[0] + s1[1] - g1_ref[...]) + b1_ref[...]
    st = st_ref[...]
    mean = st[0:1] * jnp.float32(1.0 / N)
    var = st[1:2] * jnp.float32(1.0 / N) - mean * mean
    hb = (a1 - mean) * lax.rsqrt(var + 1e-5) * gam_ref[...] + bet_ref[...]
    h = hx_ref[...] + _dot(_kan_lhs(hb),
                           _kan_rhs(w2b_ref[...], w2s_ref, w2c_ref[...]))
    h_ref[...] = h
    g_ref[...] = h * dinv[:, :40]


def _c3_body(s2_ref, g2_ref, d_ref, bo_ref, out_ref):
    s2 = s2_ref[...]
    out_ref[...] = (d_ref[...][:, :40] * (s2[0] + s2[1] - g2_ref[...])
                    + bo_ref[...])


def _full(shape):
    return pl.BlockSpec(shape, lambda i: tuple(0 for _ in shape))


def _rows(d):
    return pl.BlockSpec((BLK, d), lambda i: (i, 0))


def _rows3(d):
    return pl.BlockSpec((NC, BLK, d), lambda i: (0, i, 0))


_CNT_SPEC = _rows3(HL)


def _padded(d):
    return jax.ShapeDtypeStruct((NP_, d), jnp.float32)


def _kan1_call(x, w1b, w1s_t, w1c, w2b, w2s_t, w2c):
    return pl.pallas_call(
        _kan1_body,
        grid=(GB,),
        in_specs=[_rows(128), _full((64, 128)), _full((7, 64, 128)),
                  _full((64, 128)), _full((40, 128)), _full((7, 40, 128)),
                  _full((40, 128))],
        out_specs=[_rows(64), _rows(40)],
        out_shape=[jax.ShapeDtypeStruct((N, 64), jnp.float32),
                   jax.ShapeDtypeStruct((N, 40), jnp.float32)],
    )(x, w1b, w1s_t, w1c, w2b, w2s_t, w2c)


def _scale_call(h1, hist):
    return pl.pallas_call(
        _scale_body,
        grid=(GB,),
        in_specs=[_rows(64), _CNT_SPEC],
        out_specs=[_rows(64), _rows(64)],
        out_shape=[_padded(64), _padded(64)],
    )(h1, hist)


def _c1_call(s1, g1, hist, b1):
    return pl.pallas_call(
        _c1_body,
        grid=(GB,),
        in_specs=[_rows3(64), _rows(64), _rows(64), _full((1, 64))],
        out_specs=_full((2, 64)),
        out_shape=jax.ShapeDtypeStruct((2, 64), jnp.float32),
    )(s1, g1, hist, b1)


def _c2_call(s1, g1, hist, b1, st, gam, bet, h2x, w2b, w2s_t, w2c):
    return pl.pallas_call(
        _c2_body,
        grid=(GB,),
        in_specs=[_rows3(64), _rows(64), _rows(64), _full((1, 64)),
                  _full((2, 64)), _full((1, 64)), _full((1, 64)), _rows(40),
                  _full((40, 64)), _full((7, 40, 64)), _full((40, 64))],
        out_specs=[_rows(40), _rows(40)],
        out_shape=[jax.ShapeDtypeStruct((N, 40), jnp.float32), _padded(40)],
    )(s1, g1, hist, b1, st, gam, bet, h2x, w2b, w2s_t, w2c)


def _c3_call(s2, g2, hist, bo):
    return pl.pallas_call(
        _c3_body,
        grid=(GB,),
        in_specs=[_rows3(40), _rows(40), _rows(64), _full((1, 40))],
        out_specs=_rows(40),
        out_shape=jax.ShapeDtypeStruct((N, 40), jnp.float32),
    )(s2, g2, hist, bo)


def kernel(x, edge_index, w1_base, w1_spline, w1_scaler, grid1, b1,
           bn_gamma, bn_beta, wo_base, wo_spline, wo_scaler, grido, bo):
    ei = edge_index.astype(jnp.int32).reshape(2, NC, NS, CH, K)
    w1_sp_t = jnp.transpose(w1_spline, (2, 0, 1))
    wo_sp_t = jnp.transpose(wo_spline, (2, 0, 1))

    hist = _hist(ei)
    h1, h2x = _kan1_call(x, w1_base, w1_sp_t, w1_scaler,
                         wo_base[:, :128], wo_sp_t[:, :, :128],
                         wo_scaler[:, :128])
    g1, dinv64 = _scale_call(h1, hist)
    s1 = _prop64(ei, g1)
    st = _c1_call(s1, g1, dinv64, b1.reshape(1, 64))
    h2, g2 = _c2_call(s1, g1, dinv64, b1.reshape(1, 64), st,
                      bn_gamma.reshape(1, 64), bn_beta.reshape(1, 64), h2x,
                      wo_base[:, 128:], wo_sp_t[:, :, 128:], wo_scaler[:, 128:])
    s2 = _prop40(ei, g2)
    return _c3_call(s2, g2, dinv64, bo.reshape(1, 40))
